# baseline (speedup 1.0000x reference)
import functools

import jax
import jax.numpy as jnp
from jax import lax
from jax.experimental import pallas as pl
from jax.experimental.pallas import tpu as pltpu
from jax.experimental.pallas import tpu_sc as plsc

N = 10000
NPAD = 10240
E = 320000
D = 128
DE = 16
G = 64
L = 16

NC = 2
NS = 16
NW = NC * NS
EPW = E // NW
CHR = 125
CH = 128
NCH = EPW // CHR
CPS = 16
SLABS = NCH // CPS
SLAB_E = CPS * CHR
NPT = NPAD // NS
NBUF = 4

EB = 32768
EGRID = 10
EPAD2 = EB * EGRID


def _pad16(w):
    r, c = w.shape
    return jnp.pad(w, ((0, (-r) % L), (0, (-c) % L)))


def _padb(b):
    return jnp.pad(b, (0, (-b.shape[0]) % L)).reshape(1, L)


def _pre_node_body(x_ref, wn_ref, bm_ref, wu_ref, bu_ref, proj_ref, upd_ref):
    x = x_ref[...]
    proj_ref[...] = jnp.dot(x, wn_ref[...], preferred_element_type=jnp.float32) + bm_ref[...]
    upd_ref[...] = jnp.dot(x, wu_ref[...], preferred_element_type=jnp.float32) + bu_ref[...]


def _pre_edge_body(et_ref, wet_ref, *out_refs):
    val = jnp.dot(wet_ref[...], et_ref[...], preferred_element_type=jnp.float32)
    for f in range(L):
        out_refs[f][...] = val[f]


def _sc_body(src_hbm, dst_hbm, proj_hbm, *rest):
    epre_refs = rest[:L]
    (out_hbm, src_v, dst_v, rows_v, epre_v, stage_v, proj_sh, agg_sh,
     gsem, ssem, esem) = rest[L:]
    cid = lax.axis_index("c")
    sid = lax.axis_index("s")
    wid = cid * NS + sid
    rowbase = sid * NPT

    pltpu.sync_copy(proj_hbm.at[pl.ds(rowbase, NPT)], stage_v)
    pltpu.sync_copy(stage_v, proj_sh.at[pl.ds(rowbase, NPT)])

    def _zero(i, _):
        stage_v[i] = jnp.zeros((L,), jnp.float32)
        return 0
    lax.fori_loop(0, NPT, _zero, 0)
    pltpu.sync_copy(stage_v, agg_sh.at[pl.ds(rowbase, NPT)])

    pltpu.sync_copy(src_hbm.at[wid], src_v)
    pltpu.sync_copy(dst_hbm.at[wid], dst_v)

    plsc.subcore_barrier()

    eoff = wid * EPW
    iota16 = jnp.arange(L, dtype=jnp.int32)

    def _start_slab(s_next, par_next):
        for f in range(L):
            pltpu.async_copy(
                epre_refs[f].at[pl.ds(eoff + s_next * SLAB_E, SLAB_E)],
                epre_v.at[pl.ds((par_next * L + f) * SLAB_E, SLAB_E)], esem)

    def _wait_slab(s, par):
        for f in range(L):
            pltpu.make_async_copy(
                epre_refs[f].at[pl.ds(eoff + s * SLAB_E, SLAB_E)],
                epre_v.at[pl.ds((par * L + f) * SLAB_E, SLAB_E)], esem).wait()

    _start_slab(0, 0)
    for k in range(NBUF - 1):
        pltpu.async_copy(proj_sh.at[src_v.at[k]], rows_v.at[k], gsem)

    def _slab(s, _):
        par = lax.rem(s, 2)

        @pl.when(s + 1 < SLABS)
        def _():
            _start_slab(s + 1, 1 - par)

        _wait_slab(s, par)
        base_vec = (par * L + iota16) * SLAB_E

        def _chunk(c, _):
            g = s * CPS + c
            p = lax.rem(g, NBUF)
            pn = lax.rem(g + NBUF - 1, NBUF)

            @pl.when(g >= NBUF - 1)
            def _():
                pltpu.make_async_copy(
                    rows_v.at[pn], agg_sh.at[dst_v.at[g - (NBUF - 1)]], ssem).wait()

            @pl.when(g + NBUF - 1 < NCH)
            def _():
                pltpu.async_copy(
                    proj_sh.at[src_v.at[g + NBUF - 1]], rows_v.at[pn], gsem)

            pltpu.make_async_copy(
                proj_sh.at[src_v.at[g]], rows_v.at[p], gsem).wait()

            @plsc.parallel_loop(0, CH, unroll=8)
            def _edge(j):
                col = plsc.load_gather(epre_v, [base_vec + (c * CHR + j)])
                rows_v[p, j] = jnp.maximum(rows_v[p, j] + col, 0.0)

            pltpu.async_copy(rows_v.at[p], agg_sh.at[dst_v.at[g]], ssem, add=True)
            return 0
        lax.fori_loop(0, CPS, _chunk, 0)
        return 0
    lax.fori_loop(0, SLABS, _slab, 0)

    for g in range(NCH - (NBUF - 1), NCH):
        pltpu.make_async_copy(
            rows_v.at[g % NBUF], agg_sh.at[dst_v.at[g]], ssem).wait()

    plsc.subcore_barrier()

    pltpu.sync_copy(agg_sh.at[pl.ds(rowbase, NPT)], stage_v)
    pltpu.sync_copy(stage_v, out_hbm.at[cid, pl.ds(rowbase, NPT)])


def _head_body(parts_ref, upd_ref, batch_ref, wua_ref,
               w1_ref, b1_ref, w2_ref, b2_ref, w3_ref, b3_ref, w4_ref, b4_ref,
               out_ref):
    p = parts_ref[...]
    agg = p[0, :N] + p[1, :N]
    x1 = jnp.maximum(
        upd_ref[...][:N] + jnp.dot(agg, wua_ref[...], preferred_element_type=jnp.float32),
        0.0)
    x2 = jax.nn.sigmoid(jnp.dot(x1, w1_ref[...], preferred_element_type=jnp.float32) + b1_ref[...])
    x3 = jax.nn.sigmoid(jnp.dot(x2, w2_ref[...], preferred_element_type=jnp.float32) + b2_ref[...])
    onehot = jnp.where(
        lax.broadcasted_iota(jnp.int32, (G, N), 0) == batch_ref[...], 1.0, 0.0)
    pooled = jnp.dot(onehot, x3, preferred_element_type=jnp.float32)
    x4 = jax.nn.sigmoid(jnp.dot(pooled, w3_ref[...], preferred_element_type=jnp.float32) + b3_ref[...])
    out_ref[...] = jnp.dot(x4, w4_ref[...], preferred_element_type=jnp.float32) + b4_ref[...]


def kernel(edge_index, node_attr, edge_attr, batch,
           W_msg, b_msg, W_upd, b_upd,
           W1, b1, W2, b2, W3, b3, W4, b4):
    wn = _pad16(W_msg[:D])
    we = _pad16(W_msg[D:])
    wu_n = _pad16(W_upd[:D])
    wu_a = _pad16(W_upd[D:])
    bm = _padb(b_msg)
    bu = _padb(b_upd)
    w1, b1p = _pad16(W1), _padb(b1)
    w2, b2p = _pad16(W2), _padb(b2)
    w3, b3p = _pad16(W3), _padb(b3)
    w4, b4p = _pad16(W4), _padb(b4)

    f32 = jnp.float32
    node_attr_p = jnp.pad(node_attr, ((0, NPAD - N), (0, 0)))
    node_proj, node_upd = pl.pallas_call(
        _pre_node_body,
        out_shape=(jax.ShapeDtypeStruct((NPAD, L), f32),
                   jax.ShapeDtypeStruct((NPAD, L), f32)),
    )(node_attr_p, wn, bm, wu_n, bu)

    epre_list = pl.pallas_call(
        _pre_edge_body,
        grid=(EGRID,),
        in_specs=[pl.BlockSpec((DE, EB), lambda i: (0, i)),
                  pl.BlockSpec((L, L), lambda i: (0, 0))],
        out_specs=[pl.BlockSpec((EB,), lambda i: (i,)) for _ in range(L)],
        out_shape=[jax.ShapeDtypeStruct((EPAD2,), f32) for _ in range(L)],
    )(edge_attr.T, we.T)

    src3r = edge_index[0].reshape(NW, NCH, CHR)
    dst3r = edge_index[1].reshape(NW, NCH, CHR)
    src3 = jnp.concatenate([src3r, src3r[:, :, :CH - CHR]], axis=2)
    trash = (N + (jnp.arange(NCH)[:, None] * 8 + jnp.arange(CH - CHR)[None, :])
             % (NPAD - N)).astype(jnp.int32)
    dst3 = jnp.concatenate(
        [dst3r, jnp.broadcast_to(trash, (NW, NCH, CH - CHR))], axis=2)

    sc_call = functools.partial(
        pl.kernel,
        out_type=jax.ShapeDtypeStruct((NC, NPAD, L), f32),
        mesh=plsc.VectorSubcoreMesh(core_axis_name="c", subcore_axis_name="s"),
        scratch_types=[
            pltpu.VMEM((NCH, CH), jnp.int32),
            pltpu.VMEM((NCH, CH), jnp.int32),
            pltpu.VMEM((NBUF, CH, L), f32),
            pltpu.VMEM((2 * L * SLAB_E + CH,), f32),
            pltpu.VMEM((NPT, L), f32),
            pltpu.VMEM_SHARED((NPAD, L), f32),
            pltpu.VMEM_SHARED((NPAD, L), f32),
            pltpu.SemaphoreType.DMA,
            pltpu.SemaphoreType.DMA,
            pltpu.SemaphoreType.DMA,
        ],
        compiler_params=pltpu.CompilerParams(use_tc_tiling_on_sc=False,
                                             needs_layout_passes=False),
    )(_sc_body)
    partials = sc_call(src3, dst3, node_proj, *epre_list)

    out16 = pl.pallas_call(
        _head_body,
        out_shape=jax.ShapeDtypeStruct((G, L), f32),
    )(partials, node_upd, batch.reshape(1, N), wu_a,
      w1, b1p, w2, b2p, w3, b3p, w4, b4p)

    return out16[:, :1]


# speedup vs baseline: 1.0811x; 1.0811x over previous
import functools

import jax
import jax.numpy as jnp
from jax import lax
from jax.experimental import pallas as pl
from jax.experimental.pallas import tpu as pltpu
from jax.experimental.pallas import tpu_sc as plsc

N = 10000
NPAD = 10240
E = 320000
D = 128
DE = 16
G = 64
L = 16

NC = 2
NS = 16
NW = NC * NS
EPW = E // NW
CHR = 80
CH = 80
NCH = EPW // CHR
CPS = 25
SLABS = NCH // CPS
SLAB_E = CPS * CHR
NPT = NPAD // NS
NBUF = 4

EB = 32768
EGRID = 10
EPAD2 = EB * EGRID


def _pad16(w):
    r, c = w.shape
    return jnp.pad(w, ((0, (-r) % L), (0, (-c) % L)))


def _padb(b):
    return jnp.pad(b, (0, (-b.shape[0]) % L)).reshape(1, L)


def _pre_node_body(x_ref, wn_ref, bm_ref, wu_ref, bu_ref, upd_ref, *projf_refs):
    x = x_ref[...]
    upd_ref[...] = jnp.dot(x, wu_ref[...], preferred_element_type=jnp.float32) + bu_ref[...]
    projt = lax.dot_general(wn_ref[...], x, (((0,), (1,)), ((), ())),
                            preferred_element_type=jnp.float32)
    for f in range(L):
        projf_refs[f][...] = projt[f] + bm_ref[0, f]


def _pre_edge_body(et_ref, wet_ref, *out_refs):
    val = jnp.dot(wet_ref[...], et_ref[...], preferred_element_type=jnp.float32)
    for f in range(L):
        out_refs[f][...] = val[f]


def _sc_body(src_hbm, dst_hbm, *rest):
    projf_refs = rest[:L]
    epre_refs = rest[L:2 * L]
    (out_hbm, src_v, dst_v, rows_v, epre_v, stageT_v, stage_v, proj_sh, agg_sh,
     gsem, ssem, esem) = rest[2 * L:]
    cid = lax.axis_index("c")
    sid = lax.axis_index("s")
    wid = cid * NS + sid
    rowbase = sid * NPT
    iota16 = jnp.arange(L, dtype=jnp.int32)

    for f in range(L):
        pltpu.async_copy(projf_refs[f].at[pl.ds(rowbase, NPT)],
                         stageT_v.at[pl.ds(f * NPT, NPT)], esem)
    for f in range(L):
        pltpu.make_async_copy(projf_refs[f].at[pl.ds(rowbase, NPT)],
                              stageT_v.at[pl.ds(f * NPT, NPT)], esem).wait()
    base_tr = iota16 * NPT

    @plsc.parallel_loop(0, NPT, unroll=8)
    def _tr(j):
        stage_v[j] = plsc.load_gather(stageT_v, [base_tr + j])
    pltpu.sync_copy(stage_v, proj_sh.at[pl.ds(rowbase, NPT)])

    def _zero(i, _):
        stage_v[i] = jnp.zeros((L,), jnp.float32)
        return 0
    lax.fori_loop(0, NPT, _zero, 0)
    pltpu.sync_copy(stage_v, agg_sh.at[pl.ds(rowbase, NPT)])

    pltpu.sync_copy(src_hbm.at[wid], src_v)
    pltpu.sync_copy(dst_hbm.at[wid], dst_v)

    plsc.subcore_barrier()

    eoff = wid * EPW

    def _start_slab(s_next, par_next):
        for f in range(L):
            pltpu.async_copy(
                epre_refs[f].at[pl.ds(eoff + s_next * SLAB_E, SLAB_E)],
                epre_v.at[pl.ds((par_next * L + f) * SLAB_E, SLAB_E)], esem)

    def _wait_slab(s, par):
        for f in range(L):
            pltpu.make_async_copy(
                epre_refs[f].at[pl.ds(eoff + s * SLAB_E, SLAB_E)],
                epre_v.at[pl.ds((par * L + f) * SLAB_E, SLAB_E)], esem).wait()

    _start_slab(0, 0)
    for k in range(NBUF - 1):
        pltpu.async_copy(proj_sh.at[src_v.at[k]], rows_v.at[k], gsem)

    def _slab(s, _):
        par = lax.rem(s, 2)

        @pl.when(s + 1 < SLABS)
        def _():
            _start_slab(s + 1, 1 - par)

        _wait_slab(s, par)
        base_vec = (par * L + iota16) * SLAB_E

        def _chunk(c, _):
            g = s * CPS + c
            p = lax.rem(g, NBUF)
            pn = lax.rem(g + NBUF - 1, NBUF)

            @pl.when(g >= NBUF - 1)
            def _():
                pltpu.make_async_copy(
                    rows_v.at[pn], agg_sh.at[dst_v.at[g - (NBUF - 1)]], ssem).wait()

            @pl.when(g + NBUF - 1 < NCH)
            def _():
                pltpu.async_copy(
                    proj_sh.at[src_v.at[g + NBUF - 1]], rows_v.at[pn], gsem)

            pltpu.make_async_copy(
                proj_sh.at[src_v.at[g]], rows_v.at[p], gsem).wait()

            @plsc.parallel_loop(0, CH, unroll=8)
            def _edge(j):
                col = plsc.load_gather(epre_v, [base_vec + (c * CHR + j)])
                rows_v[p, j] = jnp.maximum(rows_v[p, j] + col, 0.0)

            pltpu.async_copy(rows_v.at[p], agg_sh.at[dst_v.at[g]], ssem, add=True)
            return 0
        lax.fori_loop(0, CPS, _chunk, 0)
        return 0
    lax.fori_loop(0, SLABS, _slab, 0)

    for g in range(NCH - (NBUF - 1), NCH):
        pltpu.make_async_copy(
            rows_v.at[g % NBUF], agg_sh.at[dst_v.at[g]], ssem).wait()

    plsc.subcore_barrier()

    pltpu.sync_copy(agg_sh.at[pl.ds(rowbase, NPT)], stage_v)
    pltpu.sync_copy(stage_v, out_hbm.at[cid, pl.ds(rowbase, NPT)])


def _head_body(parts_ref, upd_ref, batch_ref, wua_ref,
               w1_ref, b1_ref, w2_ref, b2_ref, w3_ref, b3_ref, w4_ref, b4_ref,
               out_ref):
    p = parts_ref[...]
    agg = p[0, :N] + p[1, :N]
    x1 = jnp.maximum(
        upd_ref[...][:N] + jnp.dot(agg, wua_ref[...], preferred_element_type=jnp.float32),
        0.0)
    x2 = jax.nn.sigmoid(jnp.dot(x1, w1_ref[...], preferred_element_type=jnp.float32) + b1_ref[...])
    x3 = jax.nn.sigmoid(jnp.dot(x2, w2_ref[...], preferred_element_type=jnp.float32) + b2_ref[...])
    onehot = jnp.where(
        lax.broadcasted_iota(jnp.int32, (G, N), 0) == batch_ref[...], 1.0, 0.0)
    pooled = jnp.dot(onehot, x3, preferred_element_type=jnp.float32)
    x4 = jax.nn.sigmoid(jnp.dot(pooled, w3_ref[...], preferred_element_type=jnp.float32) + b3_ref[...])
    out_ref[...] = jnp.dot(x4, w4_ref[...], preferred_element_type=jnp.float32) + b4_ref[...]


def kernel(edge_index, node_attr, edge_attr, batch,
           W_msg, b_msg, W_upd, b_upd,
           W1, b1, W2, b2, W3, b3, W4, b4):
    wn = _pad16(W_msg[:D])
    we = _pad16(W_msg[D:])
    wu_n = _pad16(W_upd[:D])
    wu_a = _pad16(W_upd[D:])
    bm = _padb(b_msg)
    bu = _padb(b_upd)
    w1, b1p = _pad16(W1), _padb(b1)
    w2, b2p = _pad16(W2), _padb(b2)
    w3, b3p = _pad16(W3), _padb(b3)
    w4, b4p = _pad16(W4), _padb(b4)

    f32 = jnp.float32
    node_attr_p = jnp.pad(node_attr, ((0, NPAD - N), (0, 0)))
    node_upd, *proj_list = pl.pallas_call(
        _pre_node_body,
        out_shape=tuple([jax.ShapeDtypeStruct((NPAD, L), f32)]
                        + [jax.ShapeDtypeStruct((NPAD,), f32) for _ in range(L)]),
    )(node_attr_p, wn, bm, wu_n, bu)

    epre_list = pl.pallas_call(
        _pre_edge_body,
        grid=(EGRID,),
        in_specs=[pl.BlockSpec((DE, EB), lambda i: (0, i)),
                  pl.BlockSpec((L, L), lambda i: (0, 0))],
        out_specs=[pl.BlockSpec((EB,), lambda i: (i,)) for _ in range(L)],
        out_shape=[jax.ShapeDtypeStruct((EPAD2,), f32) for _ in range(L)],
    )(edge_attr.T, we.T)

    src3 = edge_index[0].reshape(NW, NCH, CH)
    dst3 = edge_index[1].reshape(NW, NCH, CH)

    sc_call = functools.partial(
        pl.kernel,
        out_type=jax.ShapeDtypeStruct((NC, NPAD, L), f32),
        mesh=plsc.VectorSubcoreMesh(core_axis_name="c", subcore_axis_name="s"),
        scratch_types=[
            pltpu.VMEM((NCH, CH), jnp.int32),
            pltpu.VMEM((NCH, CH), jnp.int32),
            pltpu.VMEM((NBUF, CH, L), f32),
            pltpu.VMEM((2 * L * SLAB_E + CH,), f32),
            pltpu.VMEM((L * NPT,), f32),
            pltpu.VMEM((NPT, L), f32),
            pltpu.VMEM_SHARED((NPAD, L), f32),
            pltpu.VMEM_SHARED((NPAD, L), f32),
            pltpu.SemaphoreType.DMA,
            pltpu.SemaphoreType.DMA,
            pltpu.SemaphoreType.DMA,
        ],
        compiler_params=pltpu.CompilerParams(use_tc_tiling_on_sc=False,
                                             needs_layout_passes=False),
    )(_sc_body)
    partials = sc_call(src3, dst3, *proj_list, *epre_list)

    out16 = pl.pallas_call(
        _head_body,
        out_shape=jax.ShapeDtypeStruct((G, L), f32),
    )(partials, node_upd, batch.reshape(1, N), wu_a,
      w1, b1p, w2, b2p, w3, b3p, w4, b4p)

    return out16[:, :1]


# speedup vs baseline: 1.1730x; 1.0851x over previous
import functools

import jax
import jax.numpy as jnp
from jax import lax
from jax.experimental import pallas as pl
from jax.experimental.pallas import tpu as pltpu
from jax.experimental.pallas import tpu_sc as plsc

N = 10000
NPAD = 10240
E = 320000
D = 128
DE = 16
G = 64
L = 16

NC = 2
NS = 16
NW = NC * NS
EPW = E // NW
CHR = 80
CH = 80
NCH = EPW // CHR
CPS = 25
SLABS = NCH // CPS
SLAB_E = CPS * CHR
NPT = NPAD // NS
NBUF = 4

EB = 32768
EGRID = 10
EPAD2 = EB * EGRID


def _pad16(w):
    r, c = w.shape
    return jnp.pad(w, ((0, (-r) % L), (0, (-c) % L)))


def _padb(b):
    return jnp.pad(b, (0, (-b.shape[0]) % L)).reshape(1, L)


def _pre_node_body(x_ref, wn_ref, bm_ref, wu_ref, bu_ref, updt_ref, projt_ref):
    x = x_ref[...]
    projt = lax.dot_general(wn_ref[...], x, (((0,), (1,)), ((), ())),
                            preferred_element_type=jnp.float32)
    updt = lax.dot_general(wu_ref[...], x, (((0,), (1,)), ((), ())),
                           preferred_element_type=jnp.float32)
    for f in range(L):
        projt_ref[pl.ds(f * NPAD, NPAD)] = projt[f] + bm_ref[0, f]
        updt_ref[pl.ds(f * NPAD, NPAD)] = updt[f] + bu_ref[0, f]


def _pre_edge_body(et_ref, wet_ref, *out_refs):
    val = jnp.dot(wet_ref[...], et_ref[...], preferred_element_type=jnp.float32)
    for f in range(L):
        out_refs[f][...] = val[f]


def _sc_body(src_hbm, dst_hbm, projt_hbm, *rest):
    epre_refs = rest[:L]
    (out_hbm, src_v, dst_v, rows_v, epre_v, stageT_v, stage_v, proj_sh, agg_sh,
     gsem, ssem, esem) = rest[L:]
    cid = lax.axis_index("c")
    sid = lax.axis_index("s")
    wid = cid * NS + sid
    rowbase = sid * NPT
    iota16 = jnp.arange(L, dtype=jnp.int32)

    for f in range(L):
        pltpu.async_copy(projt_hbm.at[pl.ds(f * NPAD + rowbase, NPT)],
                         stageT_v.at[pl.ds(f * NPT, NPT)], esem)
    for f in range(L):
        pltpu.make_async_copy(projt_hbm.at[pl.ds(f * NPAD + rowbase, NPT)],
                              stageT_v.at[pl.ds(f * NPT, NPT)], esem).wait()
    base_tr = iota16 * NPT

    @plsc.parallel_loop(0, NPT, unroll=8)
    def _tr(j):
        stage_v[j] = plsc.load_gather(stageT_v, [base_tr + j])
    pltpu.sync_copy(stage_v, proj_sh.at[pl.ds(rowbase, NPT)])

    def _zero(i, _):
        stage_v[i] = jnp.zeros((L,), jnp.float32)
        return 0
    lax.fori_loop(0, NPT, _zero, 0)
    pltpu.sync_copy(stage_v, agg_sh.at[pl.ds(rowbase, NPT)])

    pltpu.sync_copy(src_hbm.at[wid], src_v)
    pltpu.sync_copy(dst_hbm.at[wid], dst_v)

    plsc.subcore_barrier()

    eoff = wid * EPW

    def _start_slab(s_next, par_next):
        for f in range(L):
            pltpu.async_copy(
                epre_refs[f].at[pl.ds(eoff + s_next * SLAB_E, SLAB_E)],
                epre_v.at[pl.ds((par_next * L + f) * SLAB_E, SLAB_E)], esem)

    def _wait_slab(s, par):
        for f in range(L):
            pltpu.make_async_copy(
                epre_refs[f].at[pl.ds(eoff + s * SLAB_E, SLAB_E)],
                epre_v.at[pl.ds((par * L + f) * SLAB_E, SLAB_E)], esem).wait()

    _start_slab(0, 0)
    for k in range(NBUF - 1):
        pltpu.async_copy(proj_sh.at[src_v.at[k]], rows_v.at[k], gsem)

    def _slab(s, _):
        par = lax.rem(s, 2)

        @pl.when(s + 1 < SLABS)
        def _():
            _start_slab(s + 1, 1 - par)

        _wait_slab(s, par)
        base_vec = (par * L + iota16) * SLAB_E

        def _chunk(c, _):
            g = s * CPS + c
            p = lax.rem(g, NBUF)
            pn = lax.rem(g + NBUF - 1, NBUF)

            @pl.when(g >= NBUF - 1)
            def _():
                pltpu.make_async_copy(
                    rows_v.at[pn], agg_sh.at[dst_v.at[g - (NBUF - 1)]], ssem).wait()

            @pl.when(g + NBUF - 1 < NCH)
            def _():
                pltpu.async_copy(
                    proj_sh.at[src_v.at[g + NBUF - 1]], rows_v.at[pn], gsem)

            pltpu.make_async_copy(
                proj_sh.at[src_v.at[g]], rows_v.at[p], gsem).wait()

            @plsc.parallel_loop(0, CH, unroll=8)
            def _edge(j):
                col = plsc.load_gather(epre_v, [base_vec + (c * CHR + j)])
                rows_v[p, j] = jnp.maximum(rows_v[p, j] + col, 0.0)

            pltpu.async_copy(rows_v.at[p], agg_sh.at[dst_v.at[g]], ssem, add=True)
            return 0
        lax.fori_loop(0, CPS, _chunk, 0)
        return 0
    lax.fori_loop(0, SLABS, _slab, 0)

    for g in range(NCH - (NBUF - 1), NCH):
        pltpu.make_async_copy(
            rows_v.at[g % NBUF], agg_sh.at[dst_v.at[g]], ssem).wait()

    plsc.subcore_barrier()

    pltpu.sync_copy(agg_sh.at[pl.ds(rowbase, NPT)], stage_v)

    @plsc.parallel_loop(0, NPT, unroll=8)
    def _trout(j):
        plsc.store_scatter(stageT_v, [base_tr + j], stage_v[j])
    for f in range(L):
        pltpu.async_copy(stageT_v.at[pl.ds(f * NPT, NPT)],
                         out_hbm.at[pl.ds((cid * L + f) * NPAD + rowbase, NPT)],
                         esem)
    for f in range(L):
        pltpu.make_async_copy(
            stageT_v.at[pl.ds(f * NPT, NPT)],
            out_hbm.at[pl.ds((cid * L + f) * NPAD + rowbase, NPT)], esem).wait()


def _head_body(parts_ref, updt_ref, batch_ref, wuat_ref,
               w1t_ref, b1c_ref, w2t_ref, b2c_ref, w3t_ref, b3c_ref,
               w4t_ref, b4c_ref, out_ref):
    agg_t = jnp.stack(
        [parts_ref[pl.ds(f * NPAD, N)] + parts_ref[pl.ds((L + f) * NPAD, N)]
         for f in range(L)])
    upd_t = jnp.stack([updt_ref[pl.ds(f * NPAD, N)] for f in range(L)])
    x1 = jnp.maximum(
        upd_t + jnp.dot(wuat_ref[...], agg_t, preferred_element_type=jnp.float32),
        0.0)
    x2 = jax.nn.sigmoid(jnp.dot(w1t_ref[...], x1, preferred_element_type=jnp.float32) + b1c_ref[...])
    x3 = jax.nn.sigmoid(jnp.dot(w2t_ref[...], x2, preferred_element_type=jnp.float32) + b2c_ref[...])
    onehot = jnp.where(
        lax.broadcasted_iota(jnp.int32, (G, N), 0) == batch_ref[...], 1.0, 0.0)
    pooled_t = lax.dot_general(x3, onehot, (((1,), (1,)), ((), ())),
                               preferred_element_type=jnp.float32)
    x4 = jax.nn.sigmoid(jnp.dot(w3t_ref[...], pooled_t, preferred_element_type=jnp.float32) + b3c_ref[...])
    out_ref[...] = jnp.dot(w4t_ref[...], x4, preferred_element_type=jnp.float32) + b4c_ref[...]


def kernel(edge_index, node_attr, edge_attr, batch,
           W_msg, b_msg, W_upd, b_upd,
           W1, b1, W2, b2, W3, b3, W4, b4):
    wn = _pad16(W_msg[:D])
    we = _pad16(W_msg[D:])
    wu_n = _pad16(W_upd[:D])
    wu_a = _pad16(W_upd[D:])
    bm = _padb(b_msg)
    bu = _padb(b_upd)
    w1, b1p = _pad16(W1), _padb(b1)
    w2, b2p = _pad16(W2), _padb(b2)
    w3, b3p = _pad16(W3), _padb(b3)
    w4, b4p = _pad16(W4), _padb(b4)

    f32 = jnp.float32
    node_attr_p = jnp.pad(node_attr, ((0, NPAD - N), (0, 0)))
    node_updt, projt = pl.pallas_call(
        _pre_node_body,
        out_shape=(jax.ShapeDtypeStruct((L * NPAD,), f32),
                   jax.ShapeDtypeStruct((L * NPAD,), f32)),
    )(node_attr_p, wn, bm, wu_n, bu)

    epre_list = pl.pallas_call(
        _pre_edge_body,
        grid=(EGRID,),
        in_specs=[pl.BlockSpec((DE, EB), lambda i: (0, i)),
                  pl.BlockSpec((L, L), lambda i: (0, 0))],
        out_specs=[pl.BlockSpec((EB,), lambda i: (i,)) for _ in range(L)],
        out_shape=[jax.ShapeDtypeStruct((EPAD2,), f32) for _ in range(L)],
    )(edge_attr.T, we.T)

    src3 = edge_index[0].reshape(NW, NCH, CH)
    dst3 = edge_index[1].reshape(NW, NCH, CH)

    sc_call = functools.partial(
        pl.kernel,
        out_type=jax.ShapeDtypeStruct((NC * L * NPAD,), f32),
        mesh=plsc.VectorSubcoreMesh(core_axis_name="c", subcore_axis_name="s"),
        scratch_types=[
            pltpu.VMEM((NCH, CH), jnp.int32),
            pltpu.VMEM((NCH, CH), jnp.int32),
            pltpu.VMEM((NBUF, CH, L), f32),
            pltpu.VMEM((2 * L * SLAB_E + CH,), f32),
            pltpu.VMEM((L * NPT,), f32),
            pltpu.VMEM((NPT, L), f32),
            pltpu.VMEM_SHARED((NPAD, L), f32),
            pltpu.VMEM_SHARED((NPAD, L), f32),
            pltpu.SemaphoreType.DMA,
            pltpu.SemaphoreType.DMA,
            pltpu.SemaphoreType.DMA,
        ],
        compiler_params=pltpu.CompilerParams(use_tc_tiling_on_sc=False,
                                             needs_layout_passes=False),
    )(_sc_body)
    partials = sc_call(src3, dst3, projt, *epre_list)

    out16t = pl.pallas_call(
        _head_body,
        out_shape=jax.ShapeDtypeStruct((L, G), f32),
    )(partials, node_updt, batch.reshape(1, N), wu_a.T,
      w1.T, b1p.T, w2.T, b2p.T, w3.T, b3p.T, w4.T, b4p.T)

    return out16t[0].reshape(G, 1)
